# baseline (speedup 1.0000x reference)
import functools

import jax
import jax.numpy as jnp
from jax import lax
from jax.experimental import pallas as pl
from jax.experimental.pallas import tpu as pltpu
from jax.experimental.pallas import tpu_sc as plsc

N, D, K = 10000, 128, 32
NC, NS = 2, 16
NW = NC * NS
NPAD = 10240
CALL_NODES = (5888, 4352)
G = 4
LANES = 16
CD = D // LANES


ROWS_PER_TILE = 624


def _sc_neigh_sum_body(h_hbm, pred_hbm, out_hbm, h_sp, pred_v, buf0, buf1,
                       out_v, sem0, sem1, *, nodes_per_w):
  groups = nodes_per_w // G
  sid = lax.axis_index("s")
  wid = sid * NC + lax.axis_index("c")
  base = wid * nodes_per_w
  pred_base = base * K

  pltpu.sync_copy(h_hbm.at[pl.ds(sid * ROWS_PER_TILE, ROWS_PER_TILE)],
                  h_sp.at[pl.ds(sid * ROWS_PER_TILE, ROWS_PER_TILE)])

  @pl.when(sid == 0)
  def _():
    tail = NS * ROWS_PER_TILE
    pltpu.sync_copy(h_hbm.at[pl.ds(tail, N - tail)],
                    h_sp.at[pl.ds(tail, N - tail)])

  pltpu.sync_copy(pred_hbm.at[pl.ds(pred_base, nodes_per_w * K)], pred_v)
  plsc.subcore_barrier()

  bufs = (buf0, buf1)
  sems = (sem0, sem1)

  def fire(gi, b):
    idx = pred_v.at[pl.ds(gi * (G * K), G * K)]
    pltpu.make_async_copy(h_sp.at[idx], bufs[b], sems[b]).start()

  def wait(gi, b):
    idx = pred_v.at[pl.ds(gi * (G * K), G * K)]
    pltpu.make_async_copy(h_sp.at[idx], bufs[b], sems[b]).wait()

  def accumulate(gi, b):
    buf = bufs[b]
    for j in range(G):
      node = gi * G + j

      def kbody(k, accs):
        row = j * K + k
        return tuple(accs[c] + buf[row, c * LANES:(c + 1) * LANES]
                     for c in range(CD))

      accs = tuple(jnp.zeros((LANES,), jnp.float32) for _ in range(CD))
      accs = lax.fori_loop(0, K, kbody, accs, unroll=4)
      for c2 in range(CD // 2):
        e_bits = lax.bitcast_convert_type(accs[2 * c2], jnp.int32)
        o_bits = lax.bitcast_convert_type(accs[2 * c2 + 1], jnp.int32)
        lo = lax.shift_right_logical(e_bits + jnp.int32(0x8000), 16)
        hi = (o_bits + jnp.int32(0x8000)) & jnp.int32(-65536)
        packed = lo | hi
        off = pl.multiple_of(node * (D // 2) + c2 * LANES, LANES)
        out_v[pl.ds(off, LANES)] = packed

  fire(0, 0)
  fire(1, 1)

  def outer(g2, carry):
    for b in range(2):
      gi = g2 + b
      wait(gi, b)

      @pl.when(gi + 2 < groups)
      def _():
        fire(gi + 2, b)

      accumulate(gi, b)
    return carry

  lax.fori_loop(0, groups // 2, lambda i, c: outer(i * 2, c), 0)

  pltpu.sync_copy(out_v,
                  out_hbm.at[pl.ds(base * (D // 2), nodes_per_w * (D // 2))])


def _sc_neigh_sum_call(h, pred_quarter, n_nodes):
  npw = n_nodes // NW
  mesh = plsc.VectorSubcoreMesh(core_axis_name="c", subcore_axis_name="s")
  return pl.kernel(
      functools.partial(_sc_neigh_sum_body, nodes_per_w=npw),
      out_type=jax.ShapeDtypeStruct((n_nodes * (D // 2),), jnp.int32),
      mesh=mesh,
      scratch_types=[
          pltpu.VMEM_SHARED((N, D), jnp.float32),
          pltpu.VMEM((npw * K,), jnp.int32),
          pltpu.VMEM((G * K, D), jnp.float32),
          pltpu.VMEM((G * K, D), jnp.float32),
          pltpu.VMEM((npw * (D // 2),), jnp.int32),
          pltpu.SemaphoreType.DMA,
          pltpu.SemaphoreType.DMA,
      ],
  )(h, pred_quarter)


CALL_BLK = (368, 256)


def _tc_dense_body(h_ref, n_ref, wsT_ref, we_ref, wo_ref, wg1T_ref, wg2T_ref,
                   bm_ref, bg_ref, gamma_ref, beta_ref, out_ref):
  hv = h_ref[:]
  vi = n_ref[:]
  ns_e = lax.bitcast_convert_type(vi << 16, jnp.float32)
  ns_o = lax.bitcast_convert_type(vi & jnp.int32(-65536), jnp.float32)
  m = (jnp.dot(hv, wsT_ref[:], preferred_element_type=jnp.float32)
       + jnp.dot(ns_e, we_ref[:], preferred_element_type=jnp.float32)
       + jnp.dot(ns_o, wo_ref[:], preferred_element_type=jnp.float32)
       + bm_ref[:])
  gp = (jnp.dot(hv, wg1T_ref[:], preferred_element_type=jnp.float32)
        + jnp.dot(m, wg2T_ref[:], preferred_element_type=jnp.float32)
        + bg_ref[:])
  gate = jax.nn.sigmoid(gp)
  v = gate * m + (1.0 - gate) * hv
  mu = jnp.mean(v, axis=1, keepdims=True)
  d = v - mu
  var = jnp.mean(d * d, axis=1, keepdims=True)
  y = gamma_ref[:] * d * lax.rsqrt(var + 1e-5) + beta_ref[:]
  out_ref[:] = jnp.maximum(y, 0.0)


def _tc_dense_quarter(blk, rows, h_arr, neigh_q, wsT, we, wo, wg1T, wg2T,
                      bm, bg, gamma, beta, prev=None, row0=0):
  h_spec = pl.BlockSpec((blk, D), lambda i: (i, 0))
  n_spec = pl.BlockSpec((blk, D // 2), lambda i: (i, 0))
  full = pl.BlockSpec((D, D), lambda i: (0, 0))
  half = pl.BlockSpec((D // 2, D), lambda i: (0, 0))
  vec = pl.BlockSpec((1, D), lambda i: (0, 0))
  blk0 = row0 // blk
  dense_specs = [h_spec, n_spec, full, half, half, full, full, vec, vec,
                 vec, vec]
  args = (h_arr, neigh_q, wsT, we, wo, wg1T, wg2T, bm, bg, gamma, beta)
  if prev is None:
    body = _tc_dense_body
    in_specs = dense_specs
    aliases = {}
  else:
    body = lambda prev_ref, *refs: _tc_dense_body(*refs)
    in_specs = [pl.BlockSpec(memory_space=pltpu.MemorySpace.HBM)]
    in_specs += dense_specs
    args = (prev,) + args
    aliases = {0: 0}
  return pl.pallas_call(
      body,
      grid=(rows // blk,),
      in_specs=in_specs,
      out_specs=pl.BlockSpec((blk, D), lambda i: (blk0 + i, 0)),
      out_shape=jax.ShapeDtypeStruct((N, D), jnp.float32),
      input_output_aliases=aliases,
  )(*args)


def kernel(g, h, topo_nodes, pred_map, W_self, b_self, W_neigh, b_neigh,
           W_gate, b_gate, ln_gamma, ln_beta):
  n1, n2 = CALL_NODES
  pred1 = pred_map[:n1].reshape(-1)
  neigh1 = _sc_neigh_sum_call(h, pred1, n1).reshape(n1, D // 2)
  pred2 = jnp.concatenate(
      [pred_map[n1:], jnp.zeros((NPAD - N, K), jnp.int32)],
      axis=0).reshape(-1)
  neigh2 = _sc_neigh_sum_call(h, pred2, n2).reshape(n2, D // 2)
  h_tail = jnp.concatenate(
      [h[n1:], jnp.zeros((NPAD - N, D), jnp.float32)], axis=0)
  wsT = W_self.T
  wnT = W_neigh.T * (1.0 / K)
  e_rows = jnp.array([32 * (r // 16) + r % 16 for r in range(D // 2)],
                     dtype=jnp.int32)
  we = wnT[e_rows, :]
  wo = wnT[e_rows + 16, :]
  wg1T = W_gate[:, :D].T
  wg2T = W_gate[:, D:].T
  bm = (b_self + b_neigh).reshape(1, D)
  bg = b_gate.reshape(1, D)
  gamma = ln_gamma.reshape(1, D)
  beta = ln_beta.reshape(1, D)
  out1 = _tc_dense_quarter(CALL_BLK[0], n1, h, neigh1, wsT, we, wo, wg1T,
                           wg2T, bm, bg, gamma, beta)
  return _tc_dense_quarter(CALL_BLK[1], n2, h_tail, neigh2, wsT, we, wo,
                           wg1T, wg2T, bm, bg, gamma, beta, prev=out1,
                           row0=n1)


# speedup vs baseline: 1.0393x; 1.0393x over previous
import functools

import jax
import jax.numpy as jnp
from jax import lax
from jax.experimental import pallas as pl
from jax.experimental.pallas import tpu as pltpu
from jax.experimental.pallas import tpu_sc as plsc

N, D, K = 10000, 128, 32
NC, NS = 2, 16
NW = NC * NS
NPAD = 10240
CALL_NODES = (5632, 4608)
G = 4
LANES = 16
CD = D // LANES


ROWS_PER_TILE = 624


def _sc_neigh_sum_body(h_hbm, pred_hbm, out_hbm, h_sp, pred_v, buf0, buf1,
                       out_v, sem0, sem1, *, nodes_per_w):
  groups = nodes_per_w // G
  sid = lax.axis_index("s")
  wid = sid * NC + lax.axis_index("c")
  base = wid * nodes_per_w
  pred_base = base * K

  pltpu.sync_copy(h_hbm.at[pl.ds(sid * ROWS_PER_TILE, ROWS_PER_TILE)],
                  h_sp.at[pl.ds(sid * ROWS_PER_TILE, ROWS_PER_TILE)])

  @pl.when(sid == 0)
  def _():
    tail = NS * ROWS_PER_TILE
    pltpu.sync_copy(h_hbm.at[pl.ds(tail, N - tail)],
                    h_sp.at[pl.ds(tail, N - tail)])

  pltpu.sync_copy(pred_hbm.at[pl.ds(pred_base, nodes_per_w * K)], pred_v)
  plsc.subcore_barrier()

  bufs = (buf0, buf1)
  sems = (sem0, sem1)

  def fire(gi, b):
    idx = pred_v.at[pl.ds(gi * (G * K), G * K)]
    pltpu.make_async_copy(h_sp.at[idx], bufs[b], sems[b]).start()

  def wait(gi, b):
    idx = pred_v.at[pl.ds(gi * (G * K), G * K)]
    pltpu.make_async_copy(h_sp.at[idx], bufs[b], sems[b]).wait()

  def accumulate(gi, b):
    buf = bufs[b]
    for j in range(G):
      node = gi * G + j

      def kbody(k, accs):
        row = j * K + k
        return tuple(accs[c] + buf[row, c * LANES:(c + 1) * LANES]
                     for c in range(CD))

      accs = tuple(jnp.zeros((LANES,), jnp.float32) for _ in range(CD))
      accs = lax.fori_loop(0, K, kbody, accs, unroll=4)
      for c2 in range(CD // 2):
        e_bits = lax.bitcast_convert_type(accs[2 * c2], jnp.int32)
        o_bits = lax.bitcast_convert_type(accs[2 * c2 + 1], jnp.int32)
        lo = lax.shift_right_logical(e_bits + jnp.int32(0x8000), 16)
        hi = (o_bits + jnp.int32(0x8000)) & jnp.int32(-65536)
        packed = lo | hi
        off = pl.multiple_of(node * (D // 2) + c2 * LANES, LANES)
        out_v[pl.ds(off, LANES)] = packed

  fire(0, 0)
  fire(1, 1)

  def outer(g2, carry):
    for b in range(2):
      gi = g2 + b
      wait(gi, b)

      @pl.when(gi + 2 < groups)
      def _():
        fire(gi + 2, b)

      accumulate(gi, b)
    return carry

  lax.fori_loop(0, groups // 2, lambda i, c: outer(i * 2, c), 0)

  pltpu.sync_copy(out_v,
                  out_hbm.at[pl.ds(base * (D // 2), nodes_per_w * (D // 2))])


def _sc_neigh_sum_call(h, pred_quarter, n_nodes):
  npw = n_nodes // NW
  mesh = plsc.VectorSubcoreMesh(core_axis_name="c", subcore_axis_name="s")
  return pl.kernel(
      functools.partial(_sc_neigh_sum_body, nodes_per_w=npw),
      out_type=jax.ShapeDtypeStruct((n_nodes * (D // 2),), jnp.int32),
      mesh=mesh,
      scratch_types=[
          pltpu.VMEM_SHARED((N, D), jnp.float32),
          pltpu.VMEM((npw * K,), jnp.int32),
          pltpu.VMEM((G * K, D), jnp.float32),
          pltpu.VMEM((G * K, D), jnp.float32),
          pltpu.VMEM((npw * (D // 2),), jnp.int32),
          pltpu.SemaphoreType.DMA,
          pltpu.SemaphoreType.DMA,
      ],
  )(h, pred_quarter)


CALL_BLK = (352, 512)


def _tc_dense_body(h_ref, n_ref, wsT_ref, we_ref, wo_ref, wg1T_ref, wg2T_ref,
                   bm_ref, bg_ref, gamma_ref, beta_ref, out_ref):
  hv = h_ref[:]
  vi = n_ref[:]
  ns_e = lax.bitcast_convert_type(vi << 16, jnp.float32)
  ns_o = lax.bitcast_convert_type(vi & jnp.int32(-65536), jnp.float32)
  m = (jnp.dot(hv, wsT_ref[:], preferred_element_type=jnp.float32)
       + jnp.dot(ns_e, we_ref[:], preferred_element_type=jnp.float32)
       + jnp.dot(ns_o, wo_ref[:], preferred_element_type=jnp.float32)
       + bm_ref[:])
  gp = (jnp.dot(hv, wg1T_ref[:], preferred_element_type=jnp.float32)
        + jnp.dot(m, wg2T_ref[:], preferred_element_type=jnp.float32)
        + bg_ref[:])
  gate = jax.nn.sigmoid(gp)
  v = gate * m + (1.0 - gate) * hv
  mu = jnp.mean(v, axis=1, keepdims=True)
  d = v - mu
  var = jnp.mean(d * d, axis=1, keepdims=True)
  y = gamma_ref[:] * d * lax.rsqrt(var + 1e-5) + beta_ref[:]
  out_ref[:] = jnp.maximum(y, 0.0)


def _tc_dense_quarter(blk, rows, h_arr, neigh_q, wsT, we, wo, wg1T, wg2T,
                      bm, bg, gamma, beta, prev=None, row0=0):
  h_spec = pl.BlockSpec((blk, D), lambda i: (i, 0))
  n_spec = pl.BlockSpec((blk, D // 2), lambda i: (i, 0))
  full = pl.BlockSpec((D, D), lambda i: (0, 0))
  half = pl.BlockSpec((D // 2, D), lambda i: (0, 0))
  vec = pl.BlockSpec((1, D), lambda i: (0, 0))
  blk0 = row0 // blk
  dense_specs = [h_spec, n_spec, full, half, half, full, full, vec, vec,
                 vec, vec]
  args = (h_arr, neigh_q, wsT, we, wo, wg1T, wg2T, bm, bg, gamma, beta)
  if prev is None:
    body = _tc_dense_body
    in_specs = dense_specs
    aliases = {}
  else:
    body = lambda prev_ref, *refs: _tc_dense_body(*refs)
    in_specs = [pl.BlockSpec(memory_space=pltpu.MemorySpace.HBM)]
    in_specs += dense_specs
    args = (prev,) + args
    aliases = {0: 0}
  return pl.pallas_call(
      body,
      grid=(rows // blk,),
      in_specs=in_specs,
      out_specs=pl.BlockSpec((blk, D), lambda i: (blk0 + i, 0)),
      out_shape=jax.ShapeDtypeStruct((N, D), jnp.float32),
      input_output_aliases=aliases,
  )(*args)


def kernel(g, h, topo_nodes, pred_map, W_self, b_self, W_neigh, b_neigh,
           W_gate, b_gate, ln_gamma, ln_beta):
  n1, n2 = CALL_NODES
  pred1 = pred_map[:n1].reshape(-1)
  neigh1 = _sc_neigh_sum_call(h, pred1, n1).reshape(n1, D // 2)
  pred2 = jnp.concatenate(
      [pred_map[n1:], jnp.zeros((NPAD - N, K), jnp.int32)],
      axis=0).reshape(-1)
  neigh2 = _sc_neigh_sum_call(h, pred2, n2).reshape(n2, D // 2)
  h_tail = jnp.concatenate(
      [h[n1:], jnp.zeros((NPAD - N, D), jnp.float32)], axis=0)
  wsT = W_self.T
  wnT = W_neigh.T * (1.0 / K)
  e_rows = jnp.array([32 * (r // 16) + r % 16 for r in range(D // 2)],
                     dtype=jnp.int32)
  we = wnT[e_rows, :]
  wo = wnT[e_rows + 16, :]
  wg1T = W_gate[:, :D].T
  wg2T = W_gate[:, D:].T
  bm = (b_self + b_neigh).reshape(1, D)
  bg = b_gate.reshape(1, D)
  gamma = ln_gamma.reshape(1, D)
  beta = ln_beta.reshape(1, D)
  out1 = _tc_dense_quarter(CALL_BLK[0], n1, h, neigh1, wsT, we, wo, wg1T,
                           wg2T, bm, bg, gamma, beta)
  return _tc_dense_quarter(CALL_BLK[1], n2, h_tail, neigh2, wsT, we, wo,
                           wg1T, wg2T, bm, bg, gamma, beta, prev=out1,
                           row0=n1)
